# Initial kernel scaffold; baseline (speedup 1.0000x reference)
import jax
import jax.numpy as jnp
from jax.experimental import pallas as pl


def kernel(x, adj, W, a1, a2, Wc, bc):
    raise NotImplementedError("write your pallas kernel here")


# speedup vs baseline: 1.6255x; 1.6255x over previous
import jax
import jax.numpy as jnp
from jax.experimental import pallas as pl
from jax.experimental.pallas import tpu as pltpu

_N = 2708
_INS = 512
_OUTS = 64
_NM = 4
_LEAK = 0.2
_NP = 2816
_BN = 256
_NEG = -1e9


def _cat_kernel(x_ref, adj_ref, W_ref, a1_ref, a2_ref, Wc_ref, bc_ref,
                out_ref, h_scr, f1_scr, f2_scr, cond_scr):
    i = pl.program_id(0)

    @pl.when(i == 0)
    def _prologue():
        x = x_ref[...]
        cond_scr[...] = (
            jnp.dot(x, Wc_ref[...], preferred_element_type=jnp.float32)
            + bc_ref[...]
        )
        for m in range(_NM):
            h = jnp.dot(x, W_ref[m], preferred_element_type=jnp.float32)
            h_scr[m] = h
            f1_scr[m] = jnp.dot(h, a1_ref[m],
                                preferred_element_type=jnp.float32)
            f2_scr[m] = jax.lax.dot_general(
                a2_ref[m], h, (((1,), (1,)), ((), ())),
                preferred_element_type=jnp.float32)

    adj_blk = adj_ref[...]
    edge = adj_blk > 0
    col_pad = jax.lax.broadcasted_iota(jnp.int32, (_BN, _NP), 1) >= _N
    row0 = i * _BN
    cond_blk = cond_scr[pl.ds(row0, _BN), :]

    for m in range(_NM):
        f1 = f1_scr[m, pl.ds(row0, _BN), :]
        e = f1 + f2_scr[m]
        e = jnp.where(e > 0, e, _LEAK * e)
        e = jnp.where(edge, e, _NEG)
        e = jnp.where(col_pad, -jnp.inf, e)
        mx = jnp.max(e, axis=-1, keepdims=True)
        p = jnp.exp(e - mx)
        s = jnp.sum(p, axis=-1, keepdims=True)
        attn = p / s
        hp = jnp.dot(attn, h_scr[m], preferred_element_type=jnp.float32)
        gamma = cond_blk[:, m][:, None]
        beta = cond_blk[:, _NM + m][:, None]
        v = gamma * hp + beta
        out_ref[:, m * _OUTS:(m + 1) * _OUTS] = jnp.where(
            v > 0, v, jnp.exp(jnp.minimum(v, 0.0)) - 1.0)


def kernel(x, adj, W, a1, a2, Wc, bc):
    xp = jnp.zeros((_NP, _INS), jnp.float32).at[:_N, :].set(x)
    adjp = jnp.zeros((_NP, _NP), jnp.int32).at[:_N, :_N].set(adj)
    a1r = a1.reshape(_NM, _OUTS, 1)
    a2r = a2.reshape(_NM, 1, _OUTS)
    bcr = bc.reshape(1, 2 * _NM)

    grid = (_NP // _BN,)
    out = pl.pallas_call(
        _cat_kernel,
        grid=grid,
        in_specs=[
            pl.BlockSpec((_NP, _INS), lambda i: (0, 0)),
            pl.BlockSpec((_BN, _NP), lambda i: (i, 0)),
            pl.BlockSpec((_NM, _INS, _OUTS), lambda i: (0, 0, 0)),
            pl.BlockSpec((_NM, _OUTS, 1), lambda i: (0, 0, 0)),
            pl.BlockSpec((_NM, 1, _OUTS), lambda i: (0, 0, 0)),
            pl.BlockSpec((_INS, 2 * _NM), lambda i: (0, 0)),
            pl.BlockSpec((1, 2 * _NM), lambda i: (0, 0)),
        ],
        out_specs=pl.BlockSpec((_BN, _NM * _OUTS), lambda i: (i, 0)),
        out_shape=jax.ShapeDtypeStruct((_NP, _NM * _OUTS), jnp.float32),
        scratch_shapes=[
            pltpu.VMEM((_NM, _NP, _OUTS), jnp.float32),
            pltpu.VMEM((_NM, _NP, 1), jnp.float32),
            pltpu.VMEM((_NM, 1, _NP), jnp.float32),
            pltpu.VMEM((_NP, 2 * _NM), jnp.float32),
        ],
    )(xp, adjp, W, a1r, a2r, Wc, bcr)
    return out[:_N, :]


# speedup vs baseline: 2.0968x; 1.2899x over previous
import jax
import jax.numpy as jnp
from jax.experimental import pallas as pl
from jax.experimental.pallas import tpu as pltpu

_N = 2708
_INS = 512
_OUTS = 64
_NM = 4
_LEAK = 0.2
_NP = 2816
_BN = 256
_HA = 128


def _cat_kernel(x_ref, adj_ref, W_ref, a1_ref, a2_ref, Wc_ref, bc_ref,
                out_ref, h_scr, f1_scr, f2_scr, mx_scr, cond_scr):
    i = pl.program_id(0)

    @pl.when(i == 0)
    def _prologue():
        x = x_ref[...]
        cond_scr[...] = (
            jnp.dot(x, Wc_ref[...], preferred_element_type=jnp.float32)
            + bc_ref[...]
        )
        for m in range(_NM):
            h = jnp.dot(x, W_ref[m], preferred_element_type=jnp.float32)
            h_scr[m, :, :_OUTS] = h
            h_scr[m, :, _OUTS:] = jnp.ones((_NP, _HA - _OUTS), jnp.float32)
            f1 = jnp.dot(h, a1_ref[m],
                         preferred_element_type=jnp.float32)
            f1_scr[m] = f1
            f2 = jax.lax.dot_general(
                a2_ref[m], h, (((1,), (1,)), ((), ())),
                preferred_element_type=jnp.float32)
            f2_scr[m] = f2
            t = f1 + jnp.max(f2)
            mx_scr[m] = jnp.maximum(t, _LEAK * t)

    adj_blk = adj_ref[...]
    adjf = adj_blk.astype(jnp.float32)
    row0 = i * _BN
    cond_blk = cond_scr[pl.ds(row0, _BN), :]

    for m in range(_NM):
        f1 = f1_scr[m, pl.ds(row0, _BN), :]
        mx = mx_scr[m, pl.ds(row0, _BN), :]
        e = f1 + f2_scr[m]
        e = jnp.maximum(e, _LEAK * e)
        p = jnp.exp(e - mx) * adjf
        hp = jnp.dot(p, h_scr[m], preferred_element_type=jnp.float32)
        s = hp[:, _OUTS:_OUTS + 1]
        gamma = cond_blk[:, m][:, None]
        beta = cond_blk[:, _NM + m][:, None]
        v = (gamma / s) * hp[:, :_OUTS] + beta
        out_ref[:, m * _OUTS:(m + 1) * _OUTS] = jnp.where(
            v > 0, v, jnp.exp(jnp.minimum(v, 0.0)) - 1.0)


def kernel(x, adj, W, a1, a2, Wc, bc):
    xp = jnp.zeros((_NP, _INS), jnp.float32).at[:_N, :].set(x)
    adjp = jnp.zeros((_NP, _NP), jnp.int32).at[:_N, :_N].set(adj)
    a1r = a1.reshape(_NM, _OUTS, 1)
    a2r = a2.reshape(_NM, 1, _OUTS)
    bcr = bc.reshape(1, 2 * _NM)

    grid = (_NP // _BN,)
    out = pl.pallas_call(
        _cat_kernel,
        grid=grid,
        in_specs=[
            pl.BlockSpec((_NP, _INS), lambda i: (0, 0)),
            pl.BlockSpec((_BN, _NP), lambda i: (i, 0)),
            pl.BlockSpec((_NM, _INS, _OUTS), lambda i: (0, 0, 0)),
            pl.BlockSpec((_NM, _OUTS, 1), lambda i: (0, 0, 0)),
            pl.BlockSpec((_NM, 1, _OUTS), lambda i: (0, 0, 0)),
            pl.BlockSpec((_INS, 2 * _NM), lambda i: (0, 0)),
            pl.BlockSpec((1, 2 * _NM), lambda i: (0, 0)),
        ],
        out_specs=pl.BlockSpec((_BN, _NM * _OUTS), lambda i: (i, 0)),
        out_shape=jax.ShapeDtypeStruct((_NP, _NM * _OUTS), jnp.float32),
        scratch_shapes=[
            pltpu.VMEM((_NM, _NP, _HA), jnp.float32),
            pltpu.VMEM((_NM, _NP, 1), jnp.float32),
            pltpu.VMEM((_NM, 1, _NP), jnp.float32),
            pltpu.VMEM((_NM, _NP, 1), jnp.float32),
            pltpu.VMEM((_NP, 2 * _NM), jnp.float32),
        ],
    )(xp, adjp, W, a1r, a2r, Wc, bcr)
    return out[:_N, :]
